# Initial kernel scaffold; baseline (speedup 1.0000x reference)
import jax
import jax.numpy as jnp
from jax.experimental import pallas as pl


def kernel(src_pts, tar_pts, scores, relScale, relInplane):
    raise NotImplementedError("write your pallas kernel here")


# speedup vs baseline: 91.5736x; 91.5736x over previous
import functools

import jax
import jax.numpy as jnp
from jax import lax
from jax.experimental import pallas as pl
from jax.experimental.pallas import tpu as pltpu
from jax.experimental.pallas import tpu_sc as plsc

_PATCH = 14.0
_THRESH2 = 25.0 + 2.0 ** -19


def _ransac_tc_kernel(sT_ref, tT_ref, sc_ref, tc_ref, score_col_ref,
                      scale_ref, cos_ref, sin_ref,
                      m_out, failed_out, gidx_out,
                      s2_s, lt_s, kio_s, jcol_s):
    n = sc_ref.shape[1]
    f32 = jnp.float32

    @pl.when(pl.program_id(0) == 0)
    def _init():
        jj0 = lax.broadcasted_iota(jnp.int32, (n, n), 0)
        ii0 = lax.broadcasted_iota(jnp.int32, (n, n), 1)
        lt_s[...] = (ii0 <= jj0).astype(f32)
        kio_s[...] = (ii0 + 1).astype(f32)
        jcol_s[...] = jj0.astype(f32)

    sxr = sT_ref[0, 0:1, :] * _PATCH
    syr = sT_ref[0, 1:2, :] * _PATCH
    txr = tT_ref[0, 0:1, :] * _PATCH
    tyr = tT_ref[0, 1:2, :] * _PATCH
    cosr = cos_ref[0]
    sinr = sin_ref[0]
    scaler = scale_ref[0]

    a00 = scaler * cosr
    a01 = -(scaler * sinr)
    a10 = scaler * sinr

    trx = txr - (a00 * sxr + a01 * syr)
    try_ = tyr - (a10 * sxr + a00 * syr)

    bf = lambda x: x.astype(jnp.bfloat16).astype(f32)
    a00v = bf(a00)
    a01v = bf(a01)
    a10v = bf(a10)

    blk = 64
    scores_row = jnp.zeros((1, n), f32)
    for rb in range(n // blk):
        rs = pl.ds(rb * blk, blk)
        sxc = bf(sc_ref[0, rs, 0:1] * _PATCH)
        syc = bf(sc_ref[0, rs, 1:2] * _PATCH)
        txc = tc_ref[0, rs, 0:1] * _PATCH
        tyc = tc_ref[0, rs, 1:2] * _PATCH
        affx = a00v * sxc + a01v * syc + trx
        affy = a10v * sxc + a00v * syc + try_
        dx = txc - affx
        dy = tyc - affy
        s2 = dx * dx + dy * dy
        s2_s[rs, :] = s2
        jj_b = rb * blk + lax.broadcasted_iota(jnp.int32, (blk, 1), 0)
        ii_b = lax.broadcasted_iota(jnp.int32, (blk, n), 1)
        inl_b = (s2 <= _THRESH2) & (ii_b != jj_b)
        sc_b = score_col_ref[0, rs, 0:1]
        scores_row = scores_row + jnp.sum(
            jnp.where(inl_b, sc_b, 0.0), axis=0, keepdims=True)

    maxv = jnp.max(scores_row)
    irow = lax.broadcasted_iota(jnp.int32, (1, n), 1)
    idx_best = jnp.min(jnp.where(scores_row == maxv, irow, n))
    sel = irow == idx_best

    a00b = jnp.sum(jnp.where(sel, a00, 0.0))
    a01b = jnp.sum(jnp.where(sel, a01, 0.0))
    a10b = jnp.sum(jnp.where(sel, a10, 0.0))
    trxb = jnp.sum(jnp.where(sel, trx, 0.0))
    tryb = jnp.sum(jnp.where(sel, try_, 0.0))

    rio = lax.broadcasted_iota(jnp.int32, (1, 3, 3), 1)
    cio = lax.broadcasted_iota(jnp.int32, (1, 3, 3), 2)
    m = jnp.zeros((1, 3, 3), f32)
    m = jnp.where((rio == 0) & (cio == 0), a00b, m)
    m = jnp.where((rio == 0) & (cio == 1), a01b, m)
    m = jnp.where((rio == 0) & (cio == 2), trxb, m)
    m = jnp.where((rio == 1) & (cio == 0), a10b, m)
    m = jnp.where((rio == 1) & (cio == 1), a00b, m)
    m = jnp.where((rio == 1) & (cio == 2), tryb, m)
    m = jnp.where((rio == 2) & (cio == 2), 1.0, m)
    m_out[...] = m
    failed_out[...] = jnp.where(maxv == 0.0, jnp.ones((1, 1, 1), f32),
                                jnp.zeros((1, 1, 1), f32))

    inl_best = (s2_s[...] <= _THRESH2).astype(f32)
    m_col = jnp.max(jnp.where(sel, inl_best, 0.0), axis=1,
                    keepdims=True)
    jcol = lax.broadcasted_iota(jnp.int32, (n, 1), 0)
    m_col = jnp.where(jcol == idx_best, 0.0, m_col)

    c_col = lax.dot_general(lt_s[...], m_col, (((1,), (0,)), ((), ())),
                            preferred_element_type=f32)
    count = jnp.max(c_col)

    g = jnp.where((c_col == kio_s[...]) & (m_col > 0.0), 1.0, 0.0)

    gidxf = jnp.sum(g * jcol_s[...], axis=0, keepdims=True)
    krow = irow.astype(f32)
    gidx = jnp.where(krow < count, gidxf, float(n)).astype(jnp.int32)
    gidx_out[0] = gidx


def _tc_stage(src_pts, tar_pts, scores, relScale, relInplane, interpret):
    b, n, _ = src_pts.shape
    f32 = jnp.float32
    sT = src_pts.transpose(0, 2, 1)
    tT = tar_pts.transpose(0, 2, 1)
    score_col = scores.reshape(b, n, 1)
    scale_row = relScale.reshape(b, 1, n)
    cos_row = relInplane[..., 0].reshape(b, 1, n)
    sin_row = relInplane[..., 1].reshape(b, 1, n)

    out_shapes = (
        jax.ShapeDtypeStruct((b, 3, 3), f32),
        jax.ShapeDtypeStruct((b, 1, 1), f32),
        jax.ShapeDtypeStruct((b, 1, n), jnp.int32),
    )
    full = lambda *dims: pl.BlockSpec((1,) + dims, lambda i: (i,) + (0,) * len(dims))
    return pl.pallas_call(
        _ransac_tc_kernel,
        grid=(b,),
        in_specs=[
            full(2, n), full(2, n), full(n, 2), full(n, 2),
            full(n, 1), full(1, n), full(1, n), full(1, n),
        ],
        out_specs=[full(3, 3), full(1, 1), full(1, n)],
        out_shape=out_shapes,
        scratch_shapes=[pltpu.VMEM((n, n), f32)] * 4,
        interpret=interpret,
    )(sT, tT, src_pts, tar_pts, score_col, scale_row, cos_row, sin_row)


def _make_sc_gather(num_rows, n, interpret):
    num_cores, num_subcores = 2, 16
    mesh = plsc.VectorSubcoreMesh(core_axis_name="c", subcore_axis_name="s",
                                  num_cores=num_cores,
                                  num_subcores=num_subcores)
    nw = num_cores * num_subcores
    npad = n + 16

    @functools.partial(
        pl.kernel,
        out_type=jax.ShapeDtypeStruct((num_rows, n), jnp.float32),
        mesh=mesh,
        scratch_types=[
            pltpu.VMEM((npad,), jnp.float32),
            pltpu.VMEM((n,), jnp.int32),
            pltpu.VMEM((n,), jnp.float32),
        ],
        compiler_params=pltpu.CompilerParams(needs_layout_passes=False),
        interpret=interpret,
    )
    def sc_gather(vals_hbm, gidx_hbm, out_hbm, vals_v, idx_v, out_v):
        wid = lax.axis_index("s") * num_cores + lax.axis_index("c")

        def do_row(row):
            sample = row // 5
            arr = row % 5
            pltpu.sync_copy(vals_hbm.at[row], vals_v.at[pl.ds(0, n)])
            fill = jnp.where(arr == 4, 0.0, -1.0)
            vals_v[pl.ds(n, 16)] = jnp.zeros((16,), jnp.float32) + fill
            pltpu.sync_copy(gidx_hbm.at[sample], idx_v)

            def chunk(c, _):
                idx = idx_v[pl.ds(c * 16, 16)]
                out_v[pl.ds(c * 16, 16)] = plsc.load_gather(vals_v, [idx])
                return 0

            lax.fori_loop(0, n // 16, chunk, 0)
            pltpu.sync_copy(out_v, out_hbm.at[row])

        for t in range((num_rows + nw - 1) // nw):
            row = wid + t * nw

            @pl.when(row < num_rows)
            def _():
                do_row(row)

    return sc_gather


@functools.partial(jax.jit, static_argnames=("interpret",))
def _run(src_pts, tar_pts, scores, relScale, relInplane, interpret=False):
    b, n, _ = src_pts.shape
    ms, failedf, gidx = _tc_stage(src_pts, tar_pts, scores, relScale,
                                  relInplane, interpret)

    vals = jnp.concatenate(
        [src_pts.transpose(0, 2, 1), tar_pts.transpose(0, 2, 1),
         scores.reshape(b, 1, n)], axis=1).reshape(b * 5, n)

    sc = _make_sc_gather(b * 5, n, interpret)
    out = sc(vals, gidx.reshape(b, n)).reshape(b, 5, n)

    isrc = out[:, 0:2, :].transpose(0, 2, 1)
    itar = out[:, 2:4, :].transpose(0, 2, 1)
    iscore = out[:, 4, :]
    return ms, failedf.reshape(b) != 0.0, isrc, itar, iscore


def kernel(src_pts, tar_pts, scores, relScale, relInplane):
    return _run(src_pts, tar_pts, scores, relScale, relInplane)
